# baseline (speedup 1.0000x reference)
import jax
import jax.numpy as jnp
from jax import lax
from jax.experimental import pallas as pl
from jax.experimental.pallas import tpu as pltpu
from jax.experimental.pallas import tpu_sc as plsc

_N = 10000
_B = 256
_DZ = 128
_DU = 64
_DH = 256

_NS = 16

_CHUNK = 80
_WCHUNKS = 8
_RPW = _WCHUNKS * _CHUNK
_TAIL_CHUNKS = _N // _CHUNK - (_NS - 1) * _WCHUNKS


def _seg_sum_body(z_hbm, bidx_hbm, out_hbm, idx_v, rows_v, zeros_v, acc_sh, sem):
    s = lax.axis_index("s")

    def work(nchunks):
        nrows = nchunks * _CHUNK
        zcp = pltpu.async_copy(
            z_hbm.at[pl.ds(s * _RPW, nrows)], rows_v.at[pl.ds(0, nrows)], sem
        )
        for j in range(nchunks):
            pltpu.sync_copy(
                bidx_hbm.at[pl.ds(s * _RPW + j * _CHUNK, _CHUNK)], idx_v.at[j]
            )
        for i in range(16):
            for j in range(_DZ // 16):
                zeros_v[i, pl.ds(j * 16, 16)] = jnp.zeros((16,), jnp.float32)
        pltpu.sync_copy(zeros_v, acc_sh.at[pl.ds(s * 16, 16)])
        plsc.subcore_barrier()
        zcp.wait()
        for j in range(nchunks):
            pltpu.sync_copy(
                rows_v.at[pl.ds(j * _CHUNK, _CHUNK)],
                acc_sh.at[idx_v.at[j]],
                add=True,
            )

    @pl.when(s < _NS - 1)
    def _full():
        work(_WCHUNKS)

    @pl.when(s == _NS - 1)
    def _tail():
        work(_TAIL_CHUNKS)

    plsc.subcore_barrier()
    pltpu.sync_copy(acc_sh.at[pl.ds(s * 16, 16)], out_hbm.at[pl.ds(s * 16, 16)])


_seg_sum = pl.kernel(
    _seg_sum_body,
    out_type=jax.ShapeDtypeStruct((_B, _DZ), jnp.float32),
    mesh=plsc.VectorSubcoreMesh(
        core_axis_name="c", subcore_axis_name="s", num_cores=1
    ),
    scratch_types=[
        pltpu.VMEM((_WCHUNKS, _CHUNK), jnp.int32),
        pltpu.VMEM((_RPW, _DZ), jnp.float32),
        pltpu.VMEM((16, _DZ), jnp.float32),
        pltpu.VMEM_SHARED((_B, _DZ), jnp.float32),
        pltpu.SemaphoreType.DMA,
    ],
)


def _gru_block(z_ref, x_ref, h_ref, wxz, whz, wxr, whr, wxh, whh,
               bxz, bhz, bxr, bhr, bxh, bhh, u_ref, wg, bg,
               out_ref, glob_ref):
    xi = jnp.concatenate([z_ref[...], x_ref[...]], axis=1)
    h = h_ref[...]
    zg = jax.nn.sigmoid(
        jnp.dot(xi, wxz[...], preferred_element_type=jnp.float32)
        + jnp.dot(h, whz[...], preferred_element_type=jnp.float32)
        + (bxz[...] + bhz[...])
    )
    rg = jax.nn.sigmoid(
        jnp.dot(xi, wxr[...], preferred_element_type=jnp.float32)
        + jnp.dot(h, whr[...], preferred_element_type=jnp.float32)
        + (bxr[...] + bhr[...])
    )
    ht = jnp.tanh(
        jnp.dot(xi, wxh[...], preferred_element_type=jnp.float32)
        + jnp.dot(h * rg, whh[...], preferred_element_type=jnp.float32)
        + (bxh[...] + bhh[...])
    )
    out_ref[...] = zg * h + (1.0 - zg) * ht

    @pl.when(pl.program_id(0) == 0)
    def _glob():
        glob_ref[...] = jax.nn.relu(
            jnp.dot(u_ref[...], wg[...], preferred_element_type=jnp.float32)
            + bg[...]
        )


_GRU_R = 1600


def _gru(z, x, h, wxz, whz, wxr, whr, wxh, whh,
         bxz, bhz, bxr, bhr, bxh, bhh, u, wg, bg):
    n = z.shape[0]
    grid = ((n + _GRU_R - 1) // _GRU_R,)
    row_spec = lambda d: pl.BlockSpec((_GRU_R, d), lambda i: (i, 0))
    fix = lambda r, c: pl.BlockSpec((r, c), lambda i: (0, 0))
    return pl.pallas_call(
        _gru_block,
        grid=grid,
        in_specs=[
            row_spec(_DZ), row_spec(_DZ), row_spec(_DH),
            fix(_DH, _DH), fix(_DH, _DH), fix(_DH, _DH),
            fix(_DH, _DH), fix(_DH, _DH), fix(_DH, _DH),
            fix(1, _DH), fix(1, _DH), fix(1, _DH),
            fix(1, _DH), fix(1, _DH), fix(1, _DH),
            fix(_B, _DU), fix(_DU, _DZ), fix(1, _DZ),
        ],
        out_specs=[row_spec(_DH), fix(_B, _DZ)],
        out_shape=[
            jax.ShapeDtypeStruct((n, _DH), jnp.float32),
            jax.ShapeDtypeStruct((_B, _DZ), jnp.float32),
        ],
        compiler_params=pltpu.CompilerParams(
            dimension_semantics=("parallel",),
        ),
    )(z, x, h, wxz, whz, wxr, whr, wxh, whh,
      bxz, bhz, bxr, bhr, bxh, bhh, u, wg, bg)


def kernel(z, u, x, edge_index, batch, batch_size, prev_h,
           W_xz, b_xz, W_hz, b_hz, W_xr, b_xr, W_hr, b_hr,
           W_xh, b_xh, W_hh, b_hh, Wg, bg):
    graph_emb = _seg_sum(z, batch)
    r2 = lambda b: b.reshape(1, -1)
    H, glob_emb = _gru(
        z, x, prev_h, W_xz, W_hz, W_xr, W_hr, W_xh, W_hh,
        r2(b_xz), r2(b_hz), r2(b_xr), r2(b_hr), r2(b_xh), r2(b_hh),
        u, Wg, r2(bg),
    )
    fused = jnp.concatenate([graph_emb, glob_emb], axis=1)
    return fused, H


# speedup vs baseline: 1.0390x; 1.0390x over previous
import jax
import jax.numpy as jnp
from jax import lax
from jax.experimental import pallas as pl
from jax.experimental.pallas import tpu as pltpu
from jax.experimental.pallas import tpu_sc as plsc

_N = 10000
_B = 256
_DZ = 128
_DU = 64
_DH = 256

_NS = 16

_CHUNK = 80
_WCHUNKS = 8
_RPW = _WCHUNKS * _CHUNK
_TAIL_CHUNKS = _N // _CHUNK - (_NS - 1) * _WCHUNKS


def _seg_sum_body(z_hbm, bidx_hbm, out_hbm, idx_v, rows_v, zeros_v, acc_sh, sem):
    s = lax.axis_index("s")

    def work(nchunks):
        nrows = nchunks * _CHUNK
        zcp = pltpu.async_copy(
            z_hbm.at[pl.ds(s * _RPW, nrows)], rows_v.at[pl.ds(0, nrows)], sem
        )
        for j in range(nchunks):
            pltpu.sync_copy(
                bidx_hbm.at[pl.ds(s * _RPW + j * _CHUNK, _CHUNK)], idx_v.at[j]
            )
        for i in range(16):
            for j in range(_DZ // 16):
                zeros_v[i, pl.ds(j * 16, 16)] = jnp.zeros((16,), jnp.float32)
        pltpu.sync_copy(zeros_v, acc_sh.at[pl.ds(s * 16, 16)])
        plsc.subcore_barrier()
        zcp.wait()
        for j in range(nchunks):
            pltpu.sync_copy(
                rows_v.at[pl.ds(j * _CHUNK, _CHUNK)],
                acc_sh.at[idx_v.at[j]],
                add=True,
            )

    @pl.when(s < _NS - 1)
    def _full():
        work(_WCHUNKS)

    @pl.when(s == _NS - 1)
    def _tail():
        work(_TAIL_CHUNKS)

    plsc.subcore_barrier()
    pltpu.sync_copy(acc_sh.at[pl.ds(s * 16, 16)], out_hbm.at[pl.ds(s * 16, 16)])


_seg_sum = pl.kernel(
    _seg_sum_body,
    out_type=jax.ShapeDtypeStruct((_B, _DZ), jnp.float32),
    mesh=plsc.VectorSubcoreMesh(
        core_axis_name="c", subcore_axis_name="s", num_cores=1
    ),
    scratch_types=[
        pltpu.VMEM((_WCHUNKS, _CHUNK), jnp.int32),
        pltpu.VMEM((_RPW, _DZ), jnp.float32),
        pltpu.VMEM((16, _DZ), jnp.float32),
        pltpu.VMEM_SHARED((_B, _DZ), jnp.float32),
        pltpu.SemaphoreType.DMA,
    ],
    compiler_params=pltpu.CompilerParams(use_tc_tiling_on_sc=True),
)


def _gru_block(z_ref, x_ref, h_ref, wxz, whz, wxr, whr, wxh, whh,
               bxz, bhz, bxr, bhr, bxh, bhh, u_ref, wg, bg,
               out_ref, glob_ref):
    xi = jnp.concatenate([z_ref[...], x_ref[...]], axis=1)
    h = h_ref[...]
    zg = jax.nn.sigmoid(
        jnp.dot(xi, wxz[...], preferred_element_type=jnp.float32)
        + jnp.dot(h, whz[...], preferred_element_type=jnp.float32)
        + (bxz[...] + bhz[...])
    )
    rg = jax.nn.sigmoid(
        jnp.dot(xi, wxr[...], preferred_element_type=jnp.float32)
        + jnp.dot(h, whr[...], preferred_element_type=jnp.float32)
        + (bxr[...] + bhr[...])
    )
    ht = jnp.tanh(
        jnp.dot(xi, wxh[...], preferred_element_type=jnp.float32)
        + jnp.dot(h * rg, whh[...], preferred_element_type=jnp.float32)
        + (bxh[...] + bhh[...])
    )
    out_ref[...] = zg * h + (1.0 - zg) * ht

    @pl.when(pl.program_id(0) == 0)
    def _glob():
        glob_ref[...] = jax.nn.relu(
            jnp.dot(u_ref[...], wg[...], preferred_element_type=jnp.float32)
            + bg[...]
        )


_GRU_R = 2000


def _gru(z, x, h, wxz, whz, wxr, whr, wxh, whh,
         bxz, bhz, bxr, bhr, bxh, bhh, u, wg, bg):
    n = z.shape[0]
    grid = ((n + _GRU_R - 1) // _GRU_R,)
    row_spec = lambda d: pl.BlockSpec((_GRU_R, d), lambda i: (i, 0))
    fix = lambda r, c: pl.BlockSpec((r, c), lambda i: (0, 0))
    return pl.pallas_call(
        _gru_block,
        grid=grid,
        in_specs=[
            row_spec(_DZ), row_spec(_DZ), row_spec(_DH),
            fix(_DH, _DH), fix(_DH, _DH), fix(_DH, _DH),
            fix(_DH, _DH), fix(_DH, _DH), fix(_DH, _DH),
            fix(1, _DH), fix(1, _DH), fix(1, _DH),
            fix(1, _DH), fix(1, _DH), fix(1, _DH),
            fix(_B, _DU), fix(_DU, _DZ), fix(1, _DZ),
        ],
        out_specs=[row_spec(_DH), fix(_B, _DZ)],
        out_shape=[
            jax.ShapeDtypeStruct((n, _DH), jnp.float32),
            jax.ShapeDtypeStruct((_B, _DZ), jnp.float32),
        ],
        compiler_params=pltpu.CompilerParams(
            dimension_semantics=("parallel",),
        ),
    )(z, x, h, wxz, whz, wxr, whr, wxh, whh,
      bxz, bhz, bxr, bhr, bxh, bhh, u, wg, bg)


def kernel(z, u, x, edge_index, batch, batch_size, prev_h,
           W_xz, b_xz, W_hz, b_hz, W_xr, b_xr, W_hr, b_hr,
           W_xh, b_xh, W_hh, b_hh, Wg, bg):
    graph_emb = _seg_sum(z, batch)
    r2 = lambda b: b.reshape(1, -1)
    H, glob_emb = _gru(
        z, x, prev_h, W_xz, W_hz, W_xr, W_hr, W_xh, W_hh,
        r2(b_xz), r2(b_hz), r2(b_xr), r2(b_hr), r2(b_xh), r2(b_hh),
        u, Wg, r2(bg),
    )
    fused = jnp.concatenate([graph_emb, glob_emb], axis=1)
    return fused, H


# speedup vs baseline: 1.0544x; 1.0149x over previous
import jax
import jax.numpy as jnp
from jax import lax
from jax.experimental import pallas as pl
from jax.experimental.pallas import tpu as pltpu
from jax.experimental.pallas import tpu_sc as plsc

_N = 10000
_B = 256
_DZ = 128
_DU = 64
_DH = 256

_NS = 16

_CHUNK = 80
_WCHUNKS = 8
_RPW = _WCHUNKS * _CHUNK
_TAIL_CHUNKS = _N // _CHUNK - (_NS - 1) * _WCHUNKS


def _seg_sum_body(z_hbm, bidx_hbm, glob_hbm, out_hbm,
                  idx_v, rows_v, zeros_v, acc_sh, sem):
    s = lax.axis_index("s")

    def work(nchunks):
        nrows = nchunks * _CHUNK
        zcp = pltpu.async_copy(
            z_hbm.at[pl.ds(s * _RPW, nrows)], rows_v.at[pl.ds(0, nrows)], sem
        )
        pltpu.sync_copy(
            glob_hbm.at[pl.ds(s * 16, 16)],
            out_hbm.at[pl.ds(s * 16, 16), pl.ds(_DZ, _DZ)],
        )
        for j in range(nchunks):
            pltpu.sync_copy(
                bidx_hbm.at[pl.ds(s * _RPW + j * _CHUNK, _CHUNK)], idx_v.at[j]
            )
        for i in range(16):
            for j in range(_DZ // 16):
                zeros_v[i, pl.ds(j * 16, 16)] = jnp.zeros((16,), jnp.float32)
        pltpu.sync_copy(zeros_v, acc_sh.at[pl.ds(s * 16, 16)])
        plsc.subcore_barrier()
        zcp.wait()
        for j in range(nchunks):
            pltpu.sync_copy(
                rows_v.at[pl.ds(j * _CHUNK, _CHUNK)],
                acc_sh.at[idx_v.at[j]],
                add=True,
            )

    @pl.when(s < _NS - 1)
    def _full():
        work(_WCHUNKS)

    @pl.when(s == _NS - 1)
    def _tail():
        work(_TAIL_CHUNKS)

    plsc.subcore_barrier()
    pltpu.sync_copy(
        acc_sh.at[pl.ds(s * 16, 16)],
        out_hbm.at[pl.ds(s * 16, 16), pl.ds(0, _DZ)],
    )


_seg_sum_fuse = pl.kernel(
    _seg_sum_body,
    out_type=jax.ShapeDtypeStruct((_B, 2 * _DZ), jnp.float32),
    mesh=plsc.VectorSubcoreMesh(
        core_axis_name="c", subcore_axis_name="s", num_cores=1
    ),
    scratch_types=[
        pltpu.VMEM((_WCHUNKS, _CHUNK), jnp.int32),
        pltpu.VMEM((_RPW, _DZ), jnp.float32),
        pltpu.VMEM((16, _DZ), jnp.float32),
        pltpu.VMEM_SHARED((_B, _DZ), jnp.float32),
        pltpu.SemaphoreType.DMA,
    ],
)


def _glob_block(u_ref, wg_ref, bg_ref, out_ref):
    out_ref[...] = jax.nn.relu(
        jnp.dot(u_ref[...], wg_ref[...], preferred_element_type=jnp.float32)
        + bg_ref[...]
    )


def _glob(u, wg, bg):
    return pl.pallas_call(
        _glob_block,
        out_shape=jax.ShapeDtypeStruct((_B, _DZ), jnp.float32),
    )(u, wg, bg)


def _gru_block(z_ref, x_ref, h_ref, wxz, whz, wxr, whr, wxh, whh,
               bxz, bhz, bxr, bhr, bxh, bhh,
               out_ref):
    xi = jnp.concatenate([z_ref[...], x_ref[...]], axis=1)
    h = h_ref[...]
    zg = jax.nn.sigmoid(
        jnp.dot(xi, wxz[...], preferred_element_type=jnp.float32)
        + jnp.dot(h, whz[...], preferred_element_type=jnp.float32)
        + (bxz[...] + bhz[...])
    )
    rg = jax.nn.sigmoid(
        jnp.dot(xi, wxr[...], preferred_element_type=jnp.float32)
        + jnp.dot(h, whr[...], preferred_element_type=jnp.float32)
        + (bxr[...] + bhr[...])
    )
    ht = jnp.tanh(
        jnp.dot(xi, wxh[...], preferred_element_type=jnp.float32)
        + jnp.dot(h * rg, whh[...], preferred_element_type=jnp.float32)
        + (bxh[...] + bhh[...])
    )
    out_ref[...] = zg * h + (1.0 - zg) * ht


_GRU_R = 2000


def _gru(z, x, h, wxz, whz, wxr, whr, wxh, whh,
         bxz, bhz, bxr, bhr, bxh, bhh):
    n = z.shape[0]
    grid = (n // _GRU_R,)
    row_spec = lambda d: pl.BlockSpec((_GRU_R, d), lambda i: (i, 0))
    fix = lambda r, c: pl.BlockSpec((r, c), lambda i: (0, 0))
    return pl.pallas_call(
        _gru_block,
        grid=grid,
        in_specs=[
            row_spec(_DZ), row_spec(_DZ), row_spec(_DH),
            fix(_DH, _DH), fix(_DH, _DH), fix(_DH, _DH),
            fix(_DH, _DH), fix(_DH, _DH), fix(_DH, _DH),
            fix(1, _DH), fix(1, _DH), fix(1, _DH),
            fix(1, _DH), fix(1, _DH), fix(1, _DH),
        ],
        out_specs=row_spec(_DH),
        out_shape=jax.ShapeDtypeStruct((n, _DH), jnp.float32),
        compiler_params=pltpu.CompilerParams(
            dimension_semantics=("parallel",),
        ),
    )(z, x, h, wxz, whz, wxr, whr, wxh, whh,
      bxz, bhz, bxr, bhr, bxh, bhh)


def kernel(z, u, x, edge_index, batch, batch_size, prev_h,
           W_xz, b_xz, W_hz, b_hz, W_xr, b_xr, W_hr, b_hr,
           W_xh, b_xh, W_hh, b_hh, Wg, bg):
    r2 = lambda b: b.reshape(1, -1)
    glob_emb = _glob(u, Wg, r2(bg))
    fused = _seg_sum_fuse(z, batch, glob_emb)
    H = _gru(
        z, x, prev_h, W_xz, W_hz, W_xr, W_hr, W_xh, W_hh,
        r2(b_xz), r2(b_hz), r2(b_xr), r2(b_hr), r2(b_xh), r2(b_hh),
    )
    return fused, H
